# Initial kernel scaffold; baseline (speedup 1.0000x reference)
import jax
import jax.numpy as jnp
from jax.experimental import pallas as pl


def kernel(features, obj_output, reg_output, anchors, boxes, W_fc, b_fc, anchor_px, field):
    raise NotImplementedError("write your pallas kernel here")


# speedup vs baseline: 25.1033x; 25.1033x over previous
import functools

import jax
import jax.numpy as jnp
from jax.experimental import pallas as pl
from jax.experimental.pallas import tpu as pltpu

M, N, C, K, R = 192, 192, 256, 5, 4096
POOL, TILES = 2, 3
S = POOL * TILES

NB1 = 512
NB2 = 128


def _k1_body(lin_ref, apx_ref, f_ref, cat_ref, anc_ref, box_ref,
             obj_ref, rpn_ref, lab_ref, xy0_ref, wxy_ref, g_scr):
    blk = pl.program_id(0)

    def gather8(it, carry):
        base = pl.multiple_of(it * 8, 8)
        rows = [cat_ref[lin_ref[blk * NB1 + base + i], 0, :] for i in range(8)]
        g_scr[pl.ds(base, 8), :] = jnp.stack(rows, axis=0)
        return carry

    jax.lax.fori_loop(0, NB1 // 8, gather8, 0)

    g30 = g_scr[:]
    anc = anc_ref[:]
    box = box_ref[:]

    aw_i = anc[:, 3:4].astype(jnp.int32)
    g = jnp.zeros((NB1, 6), jnp.float32)
    picked = jnp.zeros((NB1, 1), jnp.bool_)
    for k in range(K):
        hit = jnp.logical_and(aw_i == apx_ref[k], jnp.logical_not(picked))
        g = g + jnp.where(hit, 1.0, 0.0) * g30[:, 6 * k:6 * k + 6]
        picked = jnp.logical_or(picked, hit)

    o0, o1 = g[:, 0:1], g[:, 1:2]
    mx = jnp.maximum(o0, o1)
    e0 = jnp.exp(o0 - mx)
    e1 = jnp.exp(o1 - mx)
    rs = 1.0 / (e0 + e1)
    obj_ref[:] = jnp.concatenate([e0 * rs, e1 * rs], axis=1)

    ax, ay, aw, ah = anc[:, 0:1], anc[:, 1:2], anc[:, 2:3], anc[:, 3:4]
    cx = g[:, 2:3] * aw + ax
    cy = g[:, 3:4] * ah + ay
    w = aw * jnp.exp(g[:, 4:5])
    h = ah * jnp.exp(g[:, 5:6])
    rpn_ref[:] = jnp.concatenate([cx, cy, w, h], axis=1)

    bx, by, bw, bh = box[:, 0:1], box[:, 1:2], box[:, 2:3], box[:, 3:4]
    rw = 1.0 / w
    rh = 1.0 / h
    tx = (bx + bw * 0.5 - cx) * rw
    ty = (by + bh * 0.5 - cy) * rh
    tw = jnp.log(bw * rw)
    th = jnp.log(bh * rh)
    lab_ref[:] = jnp.concatenate([tx, ty, tw, th], axis=1)

    f = f_ref[0]
    rf = 1.0 / f
    offs = (jax.lax.broadcasted_iota(jnp.int32, (1, S), 1).astype(jnp.float32)
            + 0.5) * (1.0 / S)
    fx = (cx - w * 0.5 + offs * w) * rf - 0.5
    fy = (cy - h * 0.5 + offs * h) * rf - 0.5
    x0 = jnp.clip(jnp.floor(fx), 0.0, float(N - 2))
    y0 = jnp.clip(jnp.floor(fy), 0.0, float(M - 2))
    wx = jnp.clip(fx - x0, 0.0, 1.0)
    wy = jnp.clip(fy - y0, 0.0, 1.0)
    xy0_ref[:] = jnp.concatenate(
        [y0.astype(jnp.int32), x0.astype(jnp.int32)], axis=1)
    wxy_ref[:] = jnp.concatenate([wy, wx], axis=1)


def _k2_body(idx_ref, w_ref, f_vmem, rpn_ref, wfc_ref, b_ref,
             out_ref, pool_scr):
    blk = pl.program_id(0)

    def group8(i8, carry):
        r0 = pl.multiple_of(i8 * 8, 8)
        cell_rows = [[] for _ in range(9)]
        for j in range(8):
            off = (blk * NB2 + r0 + j) * 12
            ly = [idx_ref[off + s] * N for s in range(S)]
            lx = [idx_ref[off + S + s] for s in range(S)]
            wy = [w_ref[off + s] for s in range(S)]
            wx = [w_ref[off + S + s] for s in range(S)]
            for ty in range(TILES):
                for tx in range(TILES):
                    vals = []
                    for iy in range(POOL):
                        for ix in range(POOL):
                            sy = POOL * ty + iy
                            sx = POOL * tx + ix
                            i0 = ly[sy] + lx[sx]
                            a = f_vmem[pl.ds(i0, 2), 0, :]
                            b = f_vmem[pl.ds(i0 + N, 2), 0, :]
                            ym = a * (1.0 - wy[sy]) + b * wy[sy]
                            v = (ym[0:1, :] * (1.0 - wx[sx])
                                 + ym[1:2, :] * wx[sx])
                            vals.append(v)
                    cell_rows[ty * TILES + tx].append(
                        jnp.maximum(jnp.maximum(vals[0], vals[1]),
                                    jnp.maximum(vals[2], vals[3])))
        for c in range(9):
            pool_scr[pl.ds(r0, 8), c * C:(c + 1) * C] = jnp.concatenate(
                cell_rows[c], axis=0)
        return carry

    jax.lax.fori_loop(0, NB2 // 8, group8, 0)

    pooled = pool_scr[:]
    reg = jnp.dot(pooled, wfc_ref[:],
                  preferred_element_type=jnp.float32) + b_ref[:]
    rpn = rpn_ref[:]
    cx = reg[:, 0:1] * rpn[:, 2:3] + rpn[:, 0:1]
    cy = reg[:, 1:2] * rpn[:, 3:4] + rpn[:, 1:2]
    w = rpn[:, 2:3] * jnp.exp(reg[:, 2:3])
    h = rpn[:, 3:4] * jnp.exp(reg[:, 3:4])
    out_ref[:] = jnp.concatenate([cx, cy, w, h], axis=1)


@jax.jit
def kernel(features, obj_output, reg_output, anchors, boxes, W_fc, b_fc,
           anchor_px, field):
    f = jnp.float32(field)

    px = ((anchors[:, 0] - f / 2) / f).astype(jnp.int32)
    py = ((anchors[:, 1] - f / 2) / f).astype(jnp.int32)
    lin = (py * N + px).astype(jnp.int32)

    cat = jnp.concatenate([obj_output.reshape(M, N, K, 2),
                           reg_output.reshape(M, N, K, 4)],
                          axis=-1).reshape(M * N, 1, K * 6)
    f_arr = jnp.reshape(f, (1,))

    grid1 = pltpu.PrefetchScalarGridSpec(
        num_scalar_prefetch=3,
        grid=(R // NB1,),
        in_specs=[
            pl.BlockSpec((M * N, 1, K * 6), lambda i, *_: (0, 0, 0)),
            pl.BlockSpec((NB1, 4), lambda i, *_: (i, 0)),
            pl.BlockSpec((NB1, 4), lambda i, *_: (i, 0)),
        ],
        out_specs=[
            pl.BlockSpec((NB1, 2), lambda i, *_: (i, 0)),
            pl.BlockSpec((NB1, 4), lambda i, *_: (i, 0)),
            pl.BlockSpec((NB1, 4), lambda i, *_: (i, 0)),
            pl.BlockSpec((NB1, 2 * S), lambda i, *_: (i, 0)),
            pl.BlockSpec((NB1, 2 * S), lambda i, *_: (i, 0)),
        ],
        scratch_shapes=[pltpu.VMEM((NB1, K * 6), jnp.float32)],
    )
    obj, rpn, lab, xy0, wxy = pl.pallas_call(
        _k1_body,
        grid_spec=grid1,
        out_shape=[
            jax.ShapeDtypeStruct((R, 2), jnp.float32),
            jax.ShapeDtypeStruct((R, 4), jnp.float32),
            jax.ShapeDtypeStruct((R, 4), jnp.float32),
            jax.ShapeDtypeStruct((R, 2 * S), jnp.int32),
            jax.ShapeDtypeStruct((R, 2 * S), jnp.float32),
        ],
        compiler_params=pltpu.CompilerParams(
            dimension_semantics=("arbitrary",),
            vmem_limit_bytes=32 * 1024 * 1024,
        ),
        name="rpn_map",
    )(lin, anchor_px.astype(jnp.int32), f_arr, cat, anchors, boxes)

    f3 = features.reshape(M * N, 1, C)
    idx_flat = xy0.reshape(R * 2 * S)
    w_flat = wxy.reshape(R * 2 * S)

    grid2 = pltpu.PrefetchScalarGridSpec(
        num_scalar_prefetch=2,
        grid=(R // NB2,),
        in_specs=[
            pl.BlockSpec((M * N, 1, C), lambda i, *_: (0, 0, 0)),
            pl.BlockSpec((NB2, 4), lambda i, *_: (i, 0)),
            pl.BlockSpec((TILES * TILES * C, 4), lambda i, *_: (0, 0)),
            pl.BlockSpec((1, 4), lambda i, *_: (0, 0)),
        ],
        out_specs=pl.BlockSpec((NB2, 4), lambda i, *_: (i, 0)),
        scratch_shapes=[
            pltpu.VMEM((NB2, TILES * TILES * C), jnp.float32),
        ],
    )
    align = pl.pallas_call(
        _k2_body,
        grid_spec=grid2,
        out_shape=jax.ShapeDtypeStruct((R, 4), jnp.float32),
        compiler_params=pltpu.CompilerParams(
            dimension_semantics=("arbitrary",),
            vmem_limit_bytes=50 * 1024 * 1024,
        ),
        name="roialign_fc",
    )(idx_flat, w_flat, f3, rpn, W_fc, b_fc.reshape(1, 4))

    return jnp.concatenate([obj, rpn, align, lab], axis=1)
